# baseline (speedup 1.0000x reference)
import functools

import jax
import jax.numpy as jnp
from jax import lax
from jax.experimental import pallas as pl
from jax.experimental.pallas import tpu as pltpu
from jax.experimental.pallas import tpu_sc as plsc

VOCAB = 1000000
EMBED_DIM = 64
BATCH = 16384
FIELDS = 26

_N = BATCH * FIELDS
_NC = 2
_NS = 16
_NW = _NC * _NS
_PER_W = _N // _NW
_CHUNK = 128
_NCH = _PER_W // _CHUNK

_mesh = plsc.VectorSubcoreMesh(core_axis_name="c", subcore_axis_name="s")


@functools.partial(
    pl.kernel,
    mesh=_mesh,
    out_type=jax.ShapeDtypeStruct((_N, EMBED_DIM), jnp.float32),
    scratch_types=[
        pltpu.VMEM((_NCH, _CHUNK), jnp.int32),
        pltpu.VMEM((_CHUNK, EMBED_DIM), jnp.float32),
        pltpu.SemaphoreType.DMA,
    ],
    compiler_params=pltpu.CompilerParams(use_tc_tiling_on_sc=False),
)
def _emb_lookup(idx_hbm, table_hbm, out_hbm, idx_v, rows_v, gsem):
    wid = lax.axis_index("s") * _NC + lax.axis_index("c")
    base = wid * _PER_W
    pltpu.sync_copy(idx_hbm.at[wid], idx_v)

    def body(j, carry):
        pltpu.async_copy(table_hbm.at[idx_v.at[j]], rows_v, gsem).wait()
        pltpu.sync_copy(rows_v, out_hbm.at[pl.ds(base + j * _CHUNK, _CHUNK)])
        return carry

    lax.fori_loop(0, _NCH, body, 0)


def kernel(x, table):
    idx = x.reshape(_NW, _NCH, _CHUNK).astype(jnp.int32)
    out = _emb_lookup(idx, table)
    return out.reshape(BATCH, FIELDS, EMBED_DIM)


# speedup vs baseline: 1.0766x; 1.0766x over previous
import functools

import jax
import jax.numpy as jnp
from jax import lax
from jax.experimental import pallas as pl
from jax.experimental.pallas import tpu as pltpu
from jax.experimental.pallas import tpu_sc as plsc

VOCAB = 1000000
EMBED_DIM = 64
BATCH = 16384
FIELDS = 26

_N = BATCH * FIELDS
_NC = 2
_NS = 16
_NW = _NC * _NS
_PER_W = _N // _NW
_CHUNK = 128
_NCH = _PER_W // _CHUNK
_K = 4
_NB = _NCH // _K
_BROWS = _K * _CHUNK

_mesh = plsc.VectorSubcoreMesh(core_axis_name="c", subcore_axis_name="s")


@functools.partial(
    pl.kernel,
    mesh=_mesh,
    out_type=jax.ShapeDtypeStruct((_N, EMBED_DIM), jnp.float32),
    scratch_types=[
        pltpu.VMEM((_NCH, _CHUNK), jnp.int32),
        pltpu.VMEM((_BROWS, EMBED_DIM), jnp.float32),
        pltpu.VMEM((_BROWS, EMBED_DIM), jnp.float32),
        pltpu.SemaphoreType.DMA,
        pltpu.SemaphoreType.DMA,
        pltpu.SemaphoreType.DMA,
        pltpu.SemaphoreType.DMA,
    ],
    compiler_params=pltpu.CompilerParams(use_tc_tiling_on_sc=False),
)
def _emb_lookup(idx_hbm, table_hbm, out_hbm, idx_v, buf0, buf1, g0, g1, s0, s1):
    wid = lax.axis_index("s") * _NC + lax.axis_index("c")
    base = wid * _PER_W
    pltpu.sync_copy(idx_hbm.at[wid], idx_v)

    bufs = (buf0, buf1)
    gsems = (g0, g1)
    ssems = (s0, s1)

    def fire_gathers(g, par):
        for b in range(_K):
            j = g * _K + b
            pltpu.async_copy(
                table_hbm.at[idx_v.at[j]],
                bufs[par].at[pl.ds(b * _CHUNK, _CHUNK)],
                gsems[par],
            )

    def drain_gathers(par):
        pltpu.make_async_copy(
            table_hbm.at[pl.ds(0, _BROWS)], bufs[par], gsems[par]
        ).wait()

    def fire_store(g, par):
        pltpu.async_copy(
            bufs[par], out_hbm.at[pl.ds(base + g * _BROWS, _BROWS)], ssems[par]
        )

    def drain_store(par):
        pltpu.make_async_copy(
            bufs[par], out_hbm.at[pl.ds(base, _BROWS)], ssems[par]
        ).wait()

    fire_gathers(0, 0)

    def outer(gg, carry):
        for par in range(2):
            g = gg * 2 + par
            nxt = 1 - par
            @pl.when(g >= 1)
            def _():
                drain_store(nxt)
            @pl.when(g + 1 < _NB)
            def _():
                fire_gathers(g + 1, nxt)
            drain_gathers(par)
            fire_store(g, par)
        return carry

    lax.fori_loop(0, _NB // 2, outer, 0)
    drain_store(1)


def kernel(x, table):
    idx = x.reshape(_NW, _NCH, _CHUNK).astype(jnp.int32)
    out = _emb_lookup(idx, table)
    return out.reshape(BATCH, FIELDS, EMBED_DIM)


# speedup vs baseline: 1.0772x; 1.0006x over previous
import functools

import jax
import jax.numpy as jnp
from jax import lax
from jax.experimental import pallas as pl
from jax.experimental.pallas import tpu as pltpu
from jax.experimental.pallas import tpu_sc as plsc

VOCAB = 1000000
EMBED_DIM = 64
BATCH = 16384
FIELDS = 26

_N = BATCH * FIELDS
_NC = 2
_NS = 16
_NW = _NC * _NS
_PER_W = _N // _NW
_CHUNK = 128
_NCH = _PER_W // _CHUNK
_K = 4
_NB = _NCH // _K
_BROWS = _K * _CHUNK

_mesh = plsc.VectorSubcoreMesh(core_axis_name="c", subcore_axis_name="s")


@functools.partial(
    pl.kernel,
    mesh=_mesh,
    out_type=jax.ShapeDtypeStruct((_N, EMBED_DIM), jnp.float32),
    scratch_types=[
        pltpu.VMEM((_PER_W,), jnp.int32),
        pltpu.VMEM((_BROWS, EMBED_DIM), jnp.float32),
        pltpu.VMEM((_BROWS, EMBED_DIM), jnp.float32),
        pltpu.SemaphoreType.DMA,
        pltpu.SemaphoreType.DMA,
        pltpu.SemaphoreType.DMA,
        pltpu.SemaphoreType.DMA,
    ],
    compiler_params=pltpu.CompilerParams(use_tc_tiling_on_sc=False),
)
def _emb_lookup(idx_hbm, table_hbm, out_hbm, idx_v, buf0, buf1, g0, g1, s0, s1):
    wid = lax.axis_index("s") * _NC + lax.axis_index("c")
    base = wid * _PER_W
    pltpu.sync_copy(idx_hbm.at[pl.ds(base, _PER_W)], idx_v)

    bufs = (buf0, buf1)
    gsems = (g0, g1)
    ssems = (s0, s1)

    def fire_gathers(g, par):
        for b in range(_K):
            j = g * _K + b
            pltpu.async_copy(
                table_hbm.at[idx_v.at[pl.ds(j * _CHUNK, _CHUNK)]],
                bufs[par].at[pl.ds(b * _CHUNK, _CHUNK)],
                gsems[par],
            )

    def drain_gathers(par):
        pltpu.make_async_copy(
            table_hbm.at[pl.ds(0, _BROWS)], bufs[par], gsems[par]
        ).wait()

    def fire_store(g, par):
        pltpu.async_copy(
            bufs[par], out_hbm.at[pl.ds(base + g * _BROWS, _BROWS)], ssems[par]
        )

    def drain_store(par):
        pltpu.make_async_copy(
            bufs[par], out_hbm.at[pl.ds(base, _BROWS)], ssems[par]
        ).wait()

    fire_gathers(0, 0)

    def outer(gg, carry):
        for par in range(2):
            g = gg * 2 + par
            nxt = 1 - par
            @pl.when(g >= 1)
            def _():
                drain_store(nxt)
            @pl.when(g + 1 < _NB)
            def _():
                fire_gathers(g + 1, nxt)
            drain_gathers(par)
            fire_store(g, par)
        return carry

    lax.fori_loop(0, _NB // 2, outer, 0)
    drain_store(1)


def kernel(x, table):
    out = _emb_lookup(x.reshape(_N), table)
    return out.reshape(BATCH, FIELDS, EMBED_DIM)
